# baseline (speedup 1.0000x reference)
import functools

import jax
import jax.numpy as jnp
from jax import lax
from jax.experimental import pallas as pl
from jax.experimental.pallas import tpu as pltpu
from jax.experimental.pallas import tpu_sc as plsc

N_USERS = 1000000
N_ITEMS = 100000
EMB = 64
BATCH = 16384

NC = 2
NS = 16
NW = NC * NS
BPW = BATCH // NW
ICHUNK = 128
NCHUNK = BPW // ICHUNK


def _mf_body(u_id_hbm, i_id_hbm, ut_hbm, it_hbm, out_hbm,
             uidx_v, iidx_v, urows_v, irows_v, outb_v, sem):
    wid = lax.axis_index("s") * NC + lax.axis_index("c")

    pltpu.sync_copy(u_id_hbm.at[wid], uidx_v)
    pltpu.sync_copy(i_id_hbm.at[wid], iidx_v)

    copies = []
    for c in range(NCHUNK):
        copies.append(pltpu.async_copy(
            ut_hbm.at[uidx_v.at[c]],
            urows_v.at[pl.ds(c * ICHUNK, ICHUNK)], sem))
        copies.append(pltpu.async_copy(
            it_hbm.at[iidx_v.at[c]],
            irows_v.at[pl.ds(c * ICHUNK, ICHUNK)], sem))
    for cp in copies:
        cp.wait()

    lane = lax.iota(jnp.int32, 16)

    def body(ch, _):
        acc = jnp.zeros((16,), jnp.float32)
        for j in range(16):
            r = ch * 16 + j
            p = urows_v[r, pl.ds(0, 16)] * irows_v[r, pl.ds(0, 16)]
            p += urows_v[r, pl.ds(16, 16)] * irows_v[r, pl.ds(16, 16)]
            p += urows_v[r, pl.ds(32, 16)] * irows_v[r, pl.ds(32, 16)]
            p += urows_v[r, pl.ds(48, 16)] * irows_v[r, pl.ds(48, 16)]
            acc = jnp.where(lane == j, jnp.sum(p), acc)
        outb_v[pl.ds(ch * 16, 16)] = acc
        return 0

    lax.fori_loop(0, BPW // 16, body, 0)

    pltpu.sync_copy(outb_v, out_hbm.at[wid])


@jax.jit
def _mf(u_id, i_id, user_table, item_table):
    mesh = plsc.VectorSubcoreMesh(core_axis_name="c", subcore_axis_name="s")
    f = functools.partial(
        pl.kernel,
        out_type=jax.ShapeDtypeStruct((NW, BPW), jnp.float32),
        mesh=mesh,
        compiler_params=pltpu.CompilerParams(
            needs_layout_passes=False, use_tc_tiling_on_sc=False),
        scratch_types=[
            pltpu.VMEM((NCHUNK, ICHUNK), jnp.int32),
            pltpu.VMEM((NCHUNK, ICHUNK), jnp.int32),
            pltpu.VMEM((BPW, EMB), jnp.float32),
            pltpu.VMEM((BPW, EMB), jnp.float32),
            pltpu.VMEM((BPW,), jnp.float32),
            pltpu.SemaphoreType.DMA,
        ],
    )(_mf_body)
    out = f(u_id.reshape(NW, NCHUNK, ICHUNK).astype(jnp.int32),
            i_id.reshape(NW, NCHUNK, ICHUNK).astype(jnp.int32),
            user_table, item_table)
    return out.reshape(BATCH)


def kernel(u_id, i_id, user_table, item_table):
    return _mf(u_id, i_id, user_table, item_table)
